# baseline (speedup 1.0000x reference)
import functools

import jax
import jax.numpy as jnp
from jax import lax
from jax.experimental import pallas as pl
from jax.experimental.pallas import tpu as pltpu
from jax.experimental.pallas import tpu_sc as plsc

NUM_CORES = 2
NUM_SUBCORES = 16
NUM_WORKERS = NUM_CORES * NUM_SUBCORES
LANES = 16


def _gmf_body(factors, batch_per_worker,
              user_ids_hbm, item_ids_hbm, user_table_hbm, item_table_hbm,
              out_hbm, uidx_v, iidx_v, urows_v, irows_v, out_v, sem):
    wid = lax.axis_index("s") * NUM_CORES + lax.axis_index("c")
    base = wid * batch_per_worker

    pltpu.sync_copy(user_ids_hbm.at[pl.ds(base, batch_per_worker)], uidx_v)
    pltpu.sync_copy(item_ids_hbm.at[pl.ds(base, batch_per_worker)], iidx_v)
    cu = pltpu.async_copy(user_table_hbm.at[uidx_v], urows_v, sem)
    ci = pltpu.async_copy(item_table_hbm.at[iidx_v], irows_v, sem)
    cu.wait()
    ci.wait()

    lane = lax.iota(jnp.int32, LANES)

    def group(g, carry):
        rows = g * LANES + lane
        acc = jnp.zeros((LANES,), jnp.float32)
        for f in range(factors):
            col = jnp.full((LANES,), f, jnp.int32)
            uv = plsc.load_gather(urows_v, [rows, col])
            iv = plsc.load_gather(irows_v, [rows, col])
            acc = acc + uv * iv
        out_v[pl.ds(g * LANES, LANES)] = 1.0 / (1.0 + jnp.exp(-acc))
        return carry

    lax.fori_loop(0, batch_per_worker // LANES, group, 0)
    pltpu.sync_copy(out_v, out_hbm.at[pl.ds(base, batch_per_worker)])


def kernel(user_ids, item_ids, user_table, item_table):
    batch = user_ids.shape[0]
    factors = user_table.shape[1]
    bpw = batch // NUM_WORKERS

    mesh = plsc.VectorSubcoreMesh(
        core_axis_name="c", subcore_axis_name="s",
        num_cores=NUM_CORES, num_subcores=NUM_SUBCORES)

    run = pl.kernel(
        functools.partial(_gmf_body, factors, bpw),
        out_type=jax.ShapeDtypeStruct((batch,), jnp.float32),
        mesh=mesh,
        scratch_types=[
            pltpu.VMEM((bpw,), jnp.int32),
            pltpu.VMEM((bpw,), jnp.int32),
            pltpu.VMEM((bpw, factors), jnp.float32),
            pltpu.VMEM((bpw, factors), jnp.float32),
            pltpu.VMEM((bpw,), jnp.float32),
            pltpu.SemaphoreType.DMA,
        ],
        compiler_params=pltpu.CompilerParams(
            needs_layout_passes=False, use_tc_tiling_on_sc=False),
    )
    return run(user_ids.astype(jnp.int32), item_ids.astype(jnp.int32),
               user_table, item_table)
